# Initial kernel scaffold; baseline (speedup 1.0000x reference)
import functools

import jax
import jax.numpy as jnp
from jax import lax
from jax.experimental import pallas as pl
from jax.experimental.pallas import tpu as pltpu
from jax.experimental.pallas import tpu_sc as plsc

CLS = 1000
CLSP = 1024
FEAT = 64
NSAMP = 1_000_000
CHUNK = 160
BLOCKS = CHUNK // 16
NCHUNKS = NSAMP // CHUNK
NW = 32
NC = 2


def _sc_partials(data, label, center):
  mesh = plsc.VectorSubcoreMesh(core_axis_name="c", subcore_axis_name="s")

  @functools.partial(
      pl.kernel,
      out_type=(
          jax.ShapeDtypeStruct((NW, CLSP), jnp.float32),
          jax.ShapeDtypeStruct((NW, CLSP), jnp.float32),
      ),
      mesh=mesh,
      scratch_types=[
          pltpu.VMEM((CLS, FEAT), jnp.float32),
          pltpu.VMEM((CHUNK, FEAT), jnp.float32),
          pltpu.VMEM((CHUNK,), jnp.float32),
          pltpu.VMEM((16, CLSP), jnp.float32),
          pltpu.VMEM((16, CLSP), jnp.float32),
          pltpu.VMEM((CLSP,), jnp.float32),
          pltpu.VMEM((CLSP,), jnp.float32),
      ],
  )
  def k(data_hbm, label_hbm, center_hbm, s_out, cnt_out,
        center_v, data_v, lbl_v, s_v, cnt_v, sf_v, cf_v):
    wid = lax.axis_index("s") * NC + lax.axis_index("c")

    zero16 = jnp.zeros((16,), jnp.float32)
    ones16 = jnp.ones((16,), jnp.float32)
    iota16 = lax.iota(jnp.int32, 16)

    def zero_body(g, _):
      for j in range(16):
        s_v[j, pl.ds(g * 16, 16)] = zero16
        cnt_v[j, pl.ds(g * 16, 16)] = zero16
      return 0

    lax.fori_loop(0, CLSP // 16, zero_body, 0)

    pltpu.sync_copy(center_hbm, center_v)

    nchunks_w = (NCHUNKS - wid + NW - 1) // NW

    def chunk_body(kk, _):
      c = wid + kk * NW
      base = c * CHUNK
      pltpu.sync_copy(data_hbm.at[pl.ds(base, CHUNK)], data_v)
      pltpu.sync_copy(label_hbm.at[pl.ds(base, CHUNK)], lbl_v)

      def block_body(b, _):
        lbl = lbl_v[pl.ds(b * 16, 16)].astype(jnp.int32)
        row = b * 16 + iota16
        acc = jnp.zeros((16,), jnp.float32)
        for f in range(FEAT):
          fv = jnp.full((16,), f, jnp.int32)
          d = plsc.load_gather(data_v, [row, fv])
          ce = plsc.load_gather(center_v, [lbl, fv])
          diff = d - ce
          acc = acc + diff * diff
        dist = jnp.sqrt(acc)
        plsc.addupdate_scatter(s_v, [iota16, lbl], dist)
        plsc.addupdate_scatter(cnt_v, [iota16, lbl], ones16)
        return 0

      lax.fori_loop(0, BLOCKS, block_body, 0)
      return 0

    lax.fori_loop(0, nchunks_w, chunk_body, 0)

    def fold_body(g, _):
      s_acc = jnp.zeros((16,), jnp.float32)
      c_acc = jnp.zeros((16,), jnp.float32)
      for j in range(16):
        s_acc = s_acc + s_v[j, pl.ds(g * 16, 16)]
        c_acc = c_acc + cnt_v[j, pl.ds(g * 16, 16)]
      sf_v[pl.ds(g * 16, 16)] = s_acc
      cf_v[pl.ds(g * 16, 16)] = c_acc
      return 0

    lax.fori_loop(0, CLSP // 16, fold_body, 0)

    pltpu.sync_copy(sf_v, s_out.at[wid])
    pltpu.sync_copy(cf_v, cnt_out.at[wid])

  return k(data, label, center)


def _tc_reduce(s_part, cnt_part):
  def body(s_ref, c_ref, o_ref):
    s = jnp.sum(s_ref[...], axis=0)
    c = jnp.sum(c_ref[...], axis=0)
    o_ref[0, 0] = jnp.sum(jnp.where(c > 0.0, s / c, 0.0))

  out = pl.pallas_call(
      body,
      out_shape=jax.ShapeDtypeStruct((1, 1), jnp.float32),
  )(s_part, cnt_part)
  return out[0, 0]


def kernel(data, label, center):
  s_part, cnt_part = _sc_partials(data, label, center)
  return _tc_reduce(s_part, cnt_part)


# speedup vs baseline: 2.9565x; 2.9565x over previous
import functools

import jax
import jax.numpy as jnp
from jax import lax
from jax.experimental import pallas as pl
from jax.experimental.pallas import tpu as pltpu
from jax.experimental.pallas import tpu_sc as plsc

CLS = 1000
CLSP = 1024
FEAT = 64
NSAMP = 1_000_000
CHUNK = 160
BLOCKS = CHUNK // 16
NCHUNKS = NSAMP // CHUNK
NW = 32
NC = 2


def _sc_partials(data, label, center):
  mesh = plsc.VectorSubcoreMesh(core_axis_name="c", subcore_axis_name="s")

  @functools.partial(
      pl.kernel,
      out_type=(
          jax.ShapeDtypeStruct((NW, CLSP), jnp.float32),
          jax.ShapeDtypeStruct((NW, CLSP), jnp.float32),
      ),
      mesh=mesh,
      compiler_params=pltpu.CompilerParams(
          needs_layout_passes=False, use_tc_tiling_on_sc=False),
      scratch_types=[
          pltpu.VMEM((CLS, FEAT), jnp.float32),
          pltpu.VMEM((CHUNK, FEAT), jnp.float32),
          pltpu.VMEM((CHUNK,), jnp.float32),
          pltpu.VMEM((16, CLSP), jnp.float32),
          pltpu.VMEM((16, CLSP), jnp.float32),
          pltpu.VMEM((CLSP,), jnp.float32),
          pltpu.VMEM((CLSP,), jnp.float32),
      ],
  )
  def k(data_hbm, label_hbm, center_hbm, s_out, cnt_out,
        center_v, data_v, lbl_v, s_v, cnt_v, sf_v, cf_v):
    wid = lax.axis_index("s") * NC + lax.axis_index("c")

    zero16 = jnp.zeros((16,), jnp.float32)
    ones16 = jnp.ones((16,), jnp.float32)
    iota16 = lax.iota(jnp.int32, 16)

    def zero_body(g, _):
      for j in range(16):
        s_v[j, pl.ds(g * 16, 16)] = zero16
        cnt_v[j, pl.ds(g * 16, 16)] = zero16
      return 0

    lax.fori_loop(0, CLSP // 16, zero_body, 0)

    pltpu.sync_copy(center_hbm, center_v)

    nchunks_w = (NCHUNKS - wid + NW - 1) // NW

    def chunk_body(kk, _):
      c = wid + kk * NW
      base = c * CHUNK
      pltpu.sync_copy(data_hbm.at[pl.ds(base, CHUNK)], data_v)
      pltpu.sync_copy(label_hbm.at[pl.ds(base, CHUNK)], lbl_v)

      def block_body(b, _):
        lbl = lbl_v[pl.ds(b * 16, 16)].astype(jnp.int32)
        row = b * 16 + iota16
        acc = jnp.zeros((16,), jnp.float32)
        for f in range(FEAT):
          fv = jnp.full((16,), f, jnp.int32)
          d = plsc.load_gather(data_v, [row, fv])
          ce = plsc.load_gather(center_v, [lbl, fv])
          diff = d - ce
          acc = acc + diff * diff
        bits = plsc.bitcast(acc, jnp.int32)
        seed = plsc.bitcast(
            lax.shift_right_logical(bits, 1) + 0x1FBD1DF5, jnp.float32)
        y = seed
        for _ in range(3):
          y = 0.5 * (y + acc / y)
        dist = y
        plsc.addupdate_scatter(s_v, [iota16, lbl], dist)
        plsc.addupdate_scatter(cnt_v, [iota16, lbl], ones16)
        return 0

      lax.fori_loop(0, BLOCKS, block_body, 0)
      return 0

    lax.fori_loop(0, nchunks_w, chunk_body, 0)

    def fold_body(g, _):
      s_acc = jnp.zeros((16,), jnp.float32)
      c_acc = jnp.zeros((16,), jnp.float32)
      for j in range(16):
        s_acc = s_acc + s_v[j, pl.ds(g * 16, 16)]
        c_acc = c_acc + cnt_v[j, pl.ds(g * 16, 16)]
      sf_v[pl.ds(g * 16, 16)] = s_acc
      cf_v[pl.ds(g * 16, 16)] = c_acc
      return 0

    lax.fori_loop(0, CLSP // 16, fold_body, 0)

    pltpu.sync_copy(sf_v, s_out.at[wid])
    pltpu.sync_copy(cf_v, cnt_out.at[wid])

  return k(data, label, center)


def _tc_reduce(s_part, cnt_part):
  def body(s_ref, c_ref, o_ref):
    s = jnp.sum(s_ref[...], axis=0)
    c = jnp.sum(c_ref[...], axis=0)
    o_ref[0, 0] = jnp.sum(jnp.where(c > 0.0, s / c, 0.0))

  out = pl.pallas_call(
      body,
      out_shape=jax.ShapeDtypeStruct((1, 1), jnp.float32),
      out_specs=pl.BlockSpec(memory_space=pltpu.SMEM),
  )(s_part, cnt_part)
  return out[0, 0]


def kernel(data, label, center):
  s_part, cnt_part = _sc_partials(data, label, center)
  return _tc_reduce(s_part, cnt_part)


# speedup vs baseline: 3.4858x; 1.1790x over previous
import functools

import jax
import jax.numpy as jnp
from jax import lax
from jax.experimental import pallas as pl
from jax.experimental.pallas import tpu as pltpu
from jax.experimental.pallas import tpu_sc as plsc

CLS = 1000
CLSP = 1024
FEAT = 64
NSAMP = 1_000_000
CHUNK = 160
BLOCKS = CHUNK // 16
NCHUNKS = NSAMP // CHUNK
NW = 32
NC = 2


def _sc_partials(data, label, center):
  mesh = plsc.VectorSubcoreMesh(core_axis_name="c", subcore_axis_name="s")

  @functools.partial(
      pl.kernel,
      out_type=(
          jax.ShapeDtypeStruct((NW, CLSP), jnp.float32),
          jax.ShapeDtypeStruct((NW, CLSP), jnp.float32),
      ),
      mesh=mesh,
      compiler_params=pltpu.CompilerParams(
          needs_layout_passes=False, use_tc_tiling_on_sc=False),
      scratch_types=[
          pltpu.VMEM((CLS, FEAT), jnp.float32),
          pltpu.VMEM((2 * CHUNK, FEAT), jnp.float32),
          pltpu.VMEM((2 * CHUNK,), jnp.float32),
          pltpu.VMEM((16, CLSP), jnp.float32),
          pltpu.VMEM((16, CLSP), jnp.float32),
          pltpu.VMEM((CLSP,), jnp.float32),
          pltpu.VMEM((CLSP,), jnp.float32),
          pltpu.SemaphoreType.DMA,
          pltpu.SemaphoreType.DMA,
      ],
  )
  def k(data_hbm, label_hbm, center_hbm, s_out, cnt_out,
        center_v, data_v, lbl_v, s_v, cnt_v, sf_v, cf_v, sem0, sem1):
    wid = lax.axis_index("s") * NC + lax.axis_index("c")

    zero16 = jnp.zeros((16,), jnp.float32)
    ones16 = jnp.ones((16,), jnp.float32)
    iota16 = lax.iota(jnp.int32, 16)

    def zero_body(g, _):
      for j in range(16):
        s_v[j, pl.ds(g * 16, 16)] = zero16
        cnt_v[j, pl.ds(g * 16, 16)] = zero16
      return 0

    lax.fori_loop(0, CLSP // 16, zero_body, 0)

    pltpu.sync_copy(center_hbm, center_v)

    nchunks_w = (NCHUNKS - wid + NW - 1) // NW

    def issue(c, slot_base, sem):
      base = c * CHUNK
      pltpu.async_copy(
          data_hbm.at[pl.ds(base, CHUNK)],
          data_v.at[pl.ds(slot_base, CHUNK)], sem)
      pltpu.async_copy(
          label_hbm.at[pl.ds(base, CHUNK)],
          lbl_v.at[pl.ds(slot_base, CHUNK)], sem)

    def wait(slot_base, sem):
      pltpu.make_async_copy(
          data_hbm.at[pl.ds(0, CHUNK)],
          data_v.at[pl.ds(slot_base, CHUNK)], sem).wait()
      pltpu.make_async_copy(
          label_hbm.at[pl.ds(0, CHUNK)],
          lbl_v.at[pl.ds(slot_base, CHUNK)], sem).wait()

    issue(wid, 0, sem0)

    def chunk_body(kk, _):
      slot = kk % 2
      vbase = slot * CHUNK

      @pl.when(slot == 0)
      def _():
        wait(0, sem0)

      @pl.when(slot == 1)
      def _():
        wait(CHUNK, sem1)

      has_next = kk + 1 < nchunks_w
      c_next = wid + (kk + 1) * NW

      @pl.when(has_next & (slot == 0))
      def _():
        issue(c_next, CHUNK, sem1)

      @pl.when(has_next & (slot == 1))
      def _():
        issue(c_next, 0, sem0)

      def one_block(b):
        lbl = lbl_v[pl.ds(vbase + b * 16, 16)].astype(jnp.int32)
        row = vbase + b * 16 + iota16
        accs = [jnp.zeros((16,), jnp.float32) for _ in range(4)]
        for f in range(FEAT):
          fv = jnp.full((16,), f, jnp.int32)
          d = plsc.load_gather(data_v, [row, fv])
          ce = plsc.load_gather(center_v, [lbl, fv])
          diff = d - ce
          accs[f % 4] = accs[f % 4] + diff * diff
        acc = (accs[0] + accs[1]) + (accs[2] + accs[3])
        bits = plsc.bitcast(acc, jnp.int32)
        y = plsc.bitcast(
            0x5F3759DF - lax.shift_right_logical(bits, 1), jnp.float32)
        h = 0.5 * acc
        for _ in range(3):
          y = y * (1.5 - (h * y) * y)
        dist = acc * y
        plsc.addupdate_scatter(s_v, [iota16, lbl], dist)
        plsc.addupdate_scatter(cnt_v, [iota16, lbl], ones16)

      def block_body(b, _):
        one_block(2 * b)
        one_block(2 * b + 1)
        return 0

      lax.fori_loop(0, BLOCKS // 2, block_body, 0)
      return 0

    lax.fori_loop(0, nchunks_w, chunk_body, 0)

    def fold_body(g, _):
      s_acc = jnp.zeros((16,), jnp.float32)
      c_acc = jnp.zeros((16,), jnp.float32)
      for j in range(16):
        s_acc = s_acc + s_v[j, pl.ds(g * 16, 16)]
        c_acc = c_acc + cnt_v[j, pl.ds(g * 16, 16)]
      sf_v[pl.ds(g * 16, 16)] = s_acc
      cf_v[pl.ds(g * 16, 16)] = c_acc
      return 0

    lax.fori_loop(0, CLSP // 16, fold_body, 0)

    pltpu.sync_copy(sf_v, s_out.at[wid])
    pltpu.sync_copy(cf_v, cnt_out.at[wid])

  return k(data, label, center)


def _tc_reduce(s_part, cnt_part):
  def body(s_ref, c_ref, o_ref):
    s = jnp.sum(s_ref[...], axis=0)
    c = jnp.sum(c_ref[...], axis=0)
    o_ref[0, 0] = jnp.sum(jnp.where(c > 0.0, s / c, 0.0))

  out = pl.pallas_call(
      body,
      out_shape=jax.ShapeDtypeStruct((1, 1), jnp.float32),
      out_specs=pl.BlockSpec(memory_space=pltpu.SMEM),
  )(s_part, cnt_part)
  return out[0, 0]


def kernel(data, label, center):
  s_part, cnt_part = _sc_partials(data, label, center)
  return _tc_reduce(s_part, cnt_part)
